# baseline (speedup 1.0000x reference)
import functools

import jax
import jax.numpy as jnp
from jax import lax
from jax.experimental import pallas as pl
from jax.experimental.pallas import tpu as pltpu
from jax.experimental.pallas import tpu_sc as plsc

N_NODES = 100000
NUM_G = 1024
DIM = 300
NUM_T = 128
CNT_W = 16
DIM_P = 304
NV = DIM_P // 16

CHUNK = 112
NUM_FULL = N_NODES // CHUNK
TAIL = N_NODES - NUM_FULL * CHUNK
TAIL_BASE = NUM_FULL * CHUNK
NUM_WORKERS = 32
MAX_CHUNKS = -(-NUM_FULL // NUM_WORKERS)
G_PER_TILE = NUM_G // 16
FLAT_W = CHUNK * DIM + 16

_mesh = plsc.VectorSubcoreMesh(core_axis_name="c", subcore_axis_name="s")


@functools.partial(
    pl.kernel,
    out_type=[
        jax.ShapeDtypeStruct((2, NUM_G, DIM_P), jnp.float32),
        jax.ShapeDtypeStruct((2, NUM_G, CNT_W), jnp.float32),
    ],
    mesh=_mesh,
    compiler_params=pltpu.CompilerParams(use_tc_tiling_on_sc=False),
    scratch_types=[
        pltpu.VMEM((FLAT_W,), jnp.float32),
        pltpu.VMEM((2, CHUNK, DIM_P), jnp.float32),
        pltpu.VMEM((3, CHUNK), jnp.int32),
        pltpu.VMEM((TAIL,), jnp.int32),
        pltpu.VMEM((CHUNK, CNT_W), jnp.float32),
        pltpu.VMEM((G_PER_TILE, CNT_W), jnp.float32),
        pltpu.VMEM_SHARED((NUM_G, DIM_P), jnp.float32),
        pltpu.VMEM_SHARED((NUM_G, CNT_W), jnp.float32),
        pltpu.SemaphoreType.DMA,
        pltpu.SemaphoreType.DMA,
        pltpu.SemaphoreType.DMA,
    ],
)
def _sc_segment_sums(
    node_hbm, batch_hbm, zs_hbm, zc_hbm, ones_hbm,
    sums_out, cnts_out,
    flat, exp, idx3, idx_t, ones_v, cntb_v, acc, cnt, sem_g, sem_s0, sem_s1,
):
    c = lax.axis_index("c")
    s = lax.axis_index("s")
    wid = s * 2 + c
    row0 = s * G_PER_TILE
    n_mine = (NUM_FULL - wid + NUM_WORKERS - 1) // NUM_WORKERS

    pltpu.sync_copy(zs_hbm, exp.at[0, pl.ds(0, G_PER_TILE)])
    pltpu.sync_copy(exp.at[0, pl.ds(0, G_PER_TILE)], acc.at[pl.ds(row0, G_PER_TILE)])
    pltpu.sync_copy(zc_hbm, cntb_v)
    pltpu.sync_copy(cntb_v, cnt.at[pl.ds(row0, G_PER_TILE)])
    pltpu.sync_copy(ones_hbm, ones_v)
    plsc.subcore_barrier()

    def gather_start(j):
        cid = j * NUM_WORKERS + wid
        pltpu.async_copy(node_hbm.at[pl.ds(cid * (CHUNK * DIM), CHUNK * DIM)],
                         flat.at[pl.ds(0, CHUNK * DIM)], sem_g)
        pltpu.async_copy(batch_hbm.at[pl.ds(cid * CHUNK, CHUNK)],
                         idx3.at[lax.rem(j, 3)], sem_g)

    def gather_wait(j):
        pltpu.make_async_copy(node_hbm.at[pl.ds(0, CHUNK * DIM)],
                              flat.at[pl.ds(0, CHUNK * DIM)], sem_g).wait()
        pltpu.make_async_copy(batch_hbm.at[pl.ds(0, CHUNK)],
                              idx3.at[lax.rem(j, 3)], sem_g).wait()

    def expand(buf, nrows):
        eb = exp.at[buf]

        def erow(r, carry):
            base = r * DIM
            for k in range(NV):
                eb[r, pl.ds(16 * k, 16)] = flat[pl.ds(base + 16 * k, 16)]
            return carry

        lax.fori_loop(0, nrows, erow, 0)

    def scatter_start(j, buf, sem):
        ib = idx3.at[lax.rem(j, 3)]
        pltpu.async_copy(exp.at[buf], acc.at[ib], sem, add=True)
        pltpu.async_copy(ones_v, cnt.at[ib], sem, add=True)

    def scatter_drain(j, buf, sem):
        ib = idx3.at[lax.rem(j, 3)]
        pltpu.make_async_copy(exp.at[buf], acc.at[ib], sem).wait()
        pltpu.make_async_copy(ones_v, cnt.at[ib], sem).wait()

    sems = [sem_s0, sem_s1]

    @pl.when(n_mine > 0)
    def _():
        gather_start(0)

    def body(j, carry):
        buf = lax.rem(j, 2)

        @pl.when(j < n_mine)
        def _():
            @pl.when(j >= 2)
            def _():
                @pl.when(buf == 0)
                def _():
                    scatter_drain(j - 2, 0, sem_s0)

                @pl.when(buf == 1)
                def _():
                    scatter_drain(j - 2, 1, sem_s1)

            gather_wait(j)
            expand(buf, CHUNK)

            @pl.when(j + 1 < n_mine)
            def _():
                gather_start(j + 1)

            @pl.when(buf == 0)
            def _():
                scatter_start(j, 0, sem_s0)

            @pl.when(buf == 1)
            def _():
                scatter_start(j, 1, sem_s1)

        return carry

    lax.fori_loop(0, MAX_CHUNKS, body, 0)

    @pl.when(n_mine >= 2)
    def _():
        b2 = lax.rem(n_mine - 2, 2)

        @pl.when(b2 == 0)
        def _():
            scatter_drain(n_mine - 2, 0, sem_s0)

        @pl.when(b2 == 1)
        def _():
            scatter_drain(n_mine - 2, 1, sem_s1)

    @pl.when(n_mine >= 1)
    def _():
        b1 = lax.rem(n_mine - 1, 2)

        @pl.when(b1 == 0)
        def _():
            scatter_drain(n_mine - 1, 0, sem_s0)

        @pl.when(b1 == 1)
        def _():
            scatter_drain(n_mine - 1, 1, sem_s1)

    @pl.when(wid == NUM_WORKERS - 1)
    def _():
        pltpu.sync_copy(node_hbm.at[pl.ds(TAIL_BASE * DIM, TAIL * DIM)],
                        flat.at[pl.ds(0, TAIL * DIM)])
        pltpu.sync_copy(batch_hbm.at[pl.ds(TAIL_BASE, TAIL)], idx_t)
        expand(0, TAIL)
        pltpu.sync_copy(exp.at[0, pl.ds(0, TAIL)], acc.at[idx_t], add=True)
        pltpu.sync_copy(ones_v.at[pl.ds(0, TAIL)], cnt.at[idx_t], add=True)

    plsc.subcore_barrier()

    pltpu.sync_copy(acc.at[pl.ds(row0, G_PER_TILE)], exp.at[0, pl.ds(0, G_PER_TILE)])
    pltpu.sync_copy(exp.at[0, pl.ds(0, G_PER_TILE)], sums_out.at[c, pl.ds(row0, G_PER_TILE)])
    pltpu.sync_copy(cnt.at[pl.ds(row0, G_PER_TILE)], cntb_v)
    pltpu.sync_copy(cntb_v, cnts_out.at[c, pl.ds(row0, G_PER_TILE)])


def _tc_head_body(sums_ref, cnts_ref, w_ref, b_ref, out_ref):
    sums = sums_ref[0] + sums_ref[1]
    counts = cnts_ref[0, :, :1] + cnts_ref[1, :, :1]
    h = sums / jnp.clip(counts, 1.0, None)
    out_ref[...] = (
        lax.dot_general(
            h, w_ref[...],
            dimension_numbers=(((1,), (1,)), ((), ())),
            preferred_element_type=jnp.float32,
        )
        + b_ref[...]
    )


_tc_head = pl.pallas_call(
    _tc_head_body,
    out_shape=jax.ShapeDtypeStruct((NUM_G, NUM_T), jnp.float32),
)


@jax.jit
def kernel(node_rep, batch, W, b):
    batch32 = batch.astype(jnp.int32)
    node_flat = node_rep.reshape(-1)
    w_p = jnp.pad(W, ((0, 0), (0, DIM_P - DIM)))
    zs = jnp.zeros((G_PER_TILE, DIM_P), jnp.float32)
    zc = jnp.zeros((G_PER_TILE, CNT_W), jnp.float32)
    ones = jnp.ones((CHUNK, CNT_W), jnp.float32)
    sums2, cnts2 = _sc_segment_sums(node_flat, batch32, zs, zc, ones)
    return _tc_head(sums2, cnts2, w_p, b.reshape(1, NUM_T))


# speedup vs baseline: 1.0524x; 1.0524x over previous
import functools

import jax
import jax.numpy as jnp
from jax import lax
from jax.experimental import pallas as pl
from jax.experimental.pallas import tpu as pltpu
from jax.experimental.pallas import tpu_sc as plsc

N_NODES = 100000
NUM_G = 1024
DIM = 300
NUM_T = 128
CNT_W = 16
DIM_P = 304
NV = DIM_P // 16

CHUNK = 112
NUM_FULL = N_NODES // CHUNK
TAIL = N_NODES - NUM_FULL * CHUNK
TAIL_BASE = NUM_FULL * CHUNK
NUM_WORKERS = 32
MAX_CHUNKS = -(-NUM_FULL // NUM_WORKERS)
G_PER_TILE = NUM_G // 16
NV_FULL = DIM // 16

_mesh = plsc.VectorSubcoreMesh(core_axis_name="c", subcore_axis_name="s")


@functools.partial(
    pl.kernel,
    out_type=[
        jax.ShapeDtypeStruct((2, NUM_G, DIM_P), jnp.float32),
        jax.ShapeDtypeStruct((2, NUM_G, CNT_W), jnp.float32),
    ],
    mesh=_mesh,
    compiler_params=pltpu.CompilerParams(use_tc_tiling_on_sc=False),
    scratch_types=[
        pltpu.VMEM((CHUNK, DIM), jnp.float32),
        pltpu.VMEM((2, CHUNK, DIM_P), jnp.float32),
        pltpu.VMEM((3, CHUNK), jnp.int32),
        pltpu.VMEM((TAIL,), jnp.int32),
        pltpu.VMEM((CHUNK, CNT_W), jnp.float32),
        pltpu.VMEM((G_PER_TILE, CNT_W), jnp.float32),
        pltpu.VMEM_SHARED((NUM_G, DIM_P), jnp.float32),
        pltpu.VMEM_SHARED((NUM_G, CNT_W), jnp.float32),
        pltpu.SemaphoreType.DMA,
        pltpu.SemaphoreType.DMA,
        pltpu.SemaphoreType.DMA,
    ],
)
def _sc_segment_sums(
    node_hbm, batch_hbm, zs_hbm, zc_hbm, ones_hbm,
    sums_out, cnts_out,
    flat, exp, idx3, idx_t, ones_v, cntb_v, acc, cnt, sem_g, sem_s0, sem_s1,
):
    c = lax.axis_index("c")
    s = lax.axis_index("s")
    wid = s * 2 + c
    row0 = s * G_PER_TILE
    n_mine = (NUM_FULL - wid + NUM_WORKERS - 1) // NUM_WORKERS

    pltpu.sync_copy(zs_hbm, exp.at[0, pl.ds(0, G_PER_TILE)])
    pltpu.sync_copy(exp.at[0, pl.ds(0, G_PER_TILE)], acc.at[pl.ds(row0, G_PER_TILE)])
    pltpu.sync_copy(zc_hbm, cntb_v)
    pltpu.sync_copy(cntb_v, cnt.at[pl.ds(row0, G_PER_TILE)])
    pltpu.sync_copy(ones_hbm, ones_v)
    plsc.subcore_barrier()

    def gather_start(j):
        cid = j * NUM_WORKERS + wid
        pltpu.async_copy(node_hbm.at[pl.ds(cid * CHUNK, CHUNK)], flat, sem_g)
        pltpu.async_copy(batch_hbm.at[pl.ds(cid * CHUNK, CHUNK)],
                         idx3.at[lax.rem(j, 3)], sem_g)

    def gather_wait(j):
        pltpu.make_async_copy(node_hbm.at[pl.ds(0, CHUNK)], flat, sem_g).wait()
        pltpu.make_async_copy(batch_hbm.at[pl.ds(0, CHUNK)],
                              idx3.at[lax.rem(j, 3)], sem_g).wait()

    def expand(buf, nrows):
        eb = exp.at[buf]

        def erow(i, carry):
            r = i * 2
            for rr in (r, r + 1):
                vals = [flat[rr, pl.ds(16 * k, 16)] for k in range(NV_FULL)]
                vals.append(flat[rr, pl.ds(DIM - 16, 16)])
                for k in range(NV_FULL):
                    eb[rr, pl.ds(16 * k, 16)] = vals[k]
                eb[rr, pl.ds(DIM - 16, 16)] = vals[NV_FULL]
            return carry

        lax.fori_loop(0, nrows // 2, erow, 0)

    def scatter_start(j, buf, sem):
        ib = idx3.at[lax.rem(j, 3)]
        pltpu.async_copy(exp.at[buf], acc.at[ib], sem, add=True)
        pltpu.async_copy(ones_v, cnt.at[ib], sem, add=True)

    def scatter_drain(j, buf, sem):
        ib = idx3.at[lax.rem(j, 3)]
        pltpu.make_async_copy(exp.at[buf], acc.at[ib], sem).wait()
        pltpu.make_async_copy(ones_v, cnt.at[ib], sem).wait()

    sems = [sem_s0, sem_s1]

    @pl.when(n_mine > 0)
    def _():
        gather_start(0)

    def body(j, carry):
        buf = lax.rem(j, 2)

        @pl.when(j < n_mine)
        def _():
            @pl.when(j >= 2)
            def _():
                @pl.when(buf == 0)
                def _():
                    scatter_drain(j - 2, 0, sem_s0)

                @pl.when(buf == 1)
                def _():
                    scatter_drain(j - 2, 1, sem_s1)

            gather_wait(j)
            expand(buf, CHUNK)

            @pl.when(j + 1 < n_mine)
            def _():
                gather_start(j + 1)

            @pl.when(buf == 0)
            def _():
                scatter_start(j, 0, sem_s0)

            @pl.when(buf == 1)
            def _():
                scatter_start(j, 1, sem_s1)

        return carry

    lax.fori_loop(0, MAX_CHUNKS, body, 0)

    @pl.when(n_mine >= 2)
    def _():
        b2 = lax.rem(n_mine - 2, 2)

        @pl.when(b2 == 0)
        def _():
            scatter_drain(n_mine - 2, 0, sem_s0)

        @pl.when(b2 == 1)
        def _():
            scatter_drain(n_mine - 2, 1, sem_s1)

    @pl.when(n_mine >= 1)
    def _():
        b1 = lax.rem(n_mine - 1, 2)

        @pl.when(b1 == 0)
        def _():
            scatter_drain(n_mine - 1, 0, sem_s0)

        @pl.when(b1 == 1)
        def _():
            scatter_drain(n_mine - 1, 1, sem_s1)

    @pl.when(wid == NUM_WORKERS - 1)
    def _():
        pltpu.sync_copy(node_hbm.at[pl.ds(TAIL_BASE, TAIL)],
                        flat.at[pl.ds(0, TAIL)])
        pltpu.sync_copy(batch_hbm.at[pl.ds(TAIL_BASE, TAIL)], idx_t)
        expand(0, TAIL)
        pltpu.sync_copy(exp.at[0, pl.ds(0, TAIL)], acc.at[idx_t], add=True)
        pltpu.sync_copy(ones_v.at[pl.ds(0, TAIL)], cnt.at[idx_t], add=True)

    plsc.subcore_barrier()

    pltpu.sync_copy(acc.at[pl.ds(row0, G_PER_TILE)], exp.at[0, pl.ds(0, G_PER_TILE)])
    pltpu.sync_copy(exp.at[0, pl.ds(0, G_PER_TILE)], sums_out.at[c, pl.ds(row0, G_PER_TILE)])
    pltpu.sync_copy(cnt.at[pl.ds(row0, G_PER_TILE)], cntb_v)
    pltpu.sync_copy(cntb_v, cnts_out.at[c, pl.ds(row0, G_PER_TILE)])


def _tc_head_body(sums_ref, cnts_ref, w_ref, b_ref, out_ref):
    sums = sums_ref[0] + sums_ref[1]
    counts = cnts_ref[0, :, :1] + cnts_ref[1, :, :1]
    h = sums / jnp.clip(counts, 1.0, None)
    out_ref[...] = (
        lax.dot_general(
            h, w_ref[...],
            dimension_numbers=(((1,), (1,)), ((), ())),
            preferred_element_type=jnp.float32,
        )
        + b_ref[...]
    )


_tc_head = pl.pallas_call(
    _tc_head_body,
    out_shape=jax.ShapeDtypeStruct((NUM_G, NUM_T), jnp.float32),
)


@jax.jit
def kernel(node_rep, batch, W, b):
    batch32 = batch.astype(jnp.int32)
    w_p = jnp.pad(W, ((0, 0), (0, DIM_P - DIM)))
    zs = jnp.zeros((G_PER_TILE, DIM_P), jnp.float32)
    zc = jnp.zeros((G_PER_TILE, CNT_W), jnp.float32)
    ones = jnp.ones((CHUNK, CNT_W), jnp.float32)
    sums2, cnts2 = _sc_segment_sums(node_rep, batch32, zs, zc, ones)
    return _tc_head(sums2, cnts2, w_p, b.reshape(1, NUM_T))


# speedup vs baseline: 2.3207x; 2.2052x over previous
import functools

import jax
import jax.numpy as jnp
from jax import lax
from jax.experimental import pallas as pl
from jax.experimental.pallas import tpu as pltpu
from jax.experimental.pallas import tpu_sc as plsc

N_NODES = 100000
NUM_G = 1024
DIM = 300
NUM_T = 128
CNT_W = 16
DIM_P = 384
NVEC = 19

CHUNK = 80
NUM_CHUNKS = N_NODES // CHUNK
NUM_WORKERS = 32
MAX_CHUNKS = -(-NUM_CHUNKS // NUM_WORKERS)
G_PER_TILE = NUM_G // 16

_mesh = plsc.VectorSubcoreMesh(core_axis_name="c", subcore_axis_name="s")


PAD_BLOCK = 2000


def _tc_tile_body(x_ref, o_ref):
    for t in range(2):
        o_ref[:, t] = x_ref[:, 128 * t:128 * (t + 1)].reshape(PAD_BLOCK // 8, 8, 128)
    tail = jnp.concatenate(
        [x_ref[:, 256:DIM], jnp.zeros((PAD_BLOCK, 384 - 256 - (DIM - 256)), jnp.float32)],
        axis=1,
    )
    o_ref[:, 2] = tail.reshape(PAD_BLOCK // 8, 8, 128)


_tc_tile = pl.pallas_call(
    _tc_tile_body,
    grid=(N_NODES // PAD_BLOCK,),
    in_specs=[pl.BlockSpec((PAD_BLOCK, DIM), lambda i: (i, 0))],
    out_specs=pl.BlockSpec((PAD_BLOCK // 8, 3, 8, 128), lambda i: (i, 0, 0, 0)),
    out_shape=jax.ShapeDtypeStruct((N_NODES // 8, 3, 8, 128), jnp.float32),
)


@functools.partial(
    pl.kernel,
    out_type=[
        jax.ShapeDtypeStruct((2, NUM_G, DIM_P), jnp.float32),
        jax.ShapeDtypeStruct((2, NUM_G, CNT_W), jnp.float32),
    ],
    mesh=_mesh,
    compiler_params=pltpu.CompilerParams(use_tc_tiling_on_sc=False),
    scratch_types=[
        pltpu.VMEM((CHUNK // 8, 3, 8, 128), jnp.float32),
        pltpu.VMEM((2, CHUNK, DIM_P), jnp.float32),
        pltpu.VMEM((3, CHUNK), jnp.int32),
        pltpu.VMEM((CHUNK, CNT_W), jnp.float32),
        pltpu.VMEM((G_PER_TILE, CNT_W), jnp.float32),
        pltpu.VMEM_SHARED((NUM_G, DIM_P), jnp.float32),
        pltpu.VMEM_SHARED((NUM_G, CNT_W), jnp.float32),
        pltpu.SemaphoreType.DMA,
        pltpu.SemaphoreType.DMA,
        pltpu.SemaphoreType.DMA,
    ],
)
def _sc_segment_sums(
    node_hbm, batch_hbm, zs_hbm, zc_hbm, ones_hbm,
    sums_out, cnts_out,
    stg, exp, idx3, ones_v, cntb_v, acc, cnt, sem_g, sem_s0, sem_s1,
):
    c = lax.axis_index("c")
    s = lax.axis_index("s")
    wid = s * 2 + c
    row0 = s * G_PER_TILE
    n_mine = (NUM_CHUNKS - wid + NUM_WORKERS - 1) // NUM_WORKERS

    pltpu.sync_copy(zs_hbm, exp.at[0, pl.ds(0, G_PER_TILE)])
    pltpu.sync_copy(exp.at[0, pl.ds(0, G_PER_TILE)], acc.at[pl.ds(row0, G_PER_TILE)])
    pltpu.sync_copy(zc_hbm, cntb_v)
    pltpu.sync_copy(cntb_v, cnt.at[pl.ds(row0, G_PER_TILE)])
    pltpu.sync_copy(ones_hbm, ones_v)
    plsc.subcore_barrier()

    def gather_start(j):
        cid = j * NUM_WORKERS + wid
        pltpu.async_copy(node_hbm.at[pl.ds(cid * (CHUNK // 8), CHUNK // 8)], stg, sem_g)
        pltpu.async_copy(batch_hbm.at[pl.ds(cid * CHUNK, CHUNK)],
                         idx3.at[lax.rem(j, 3)], sem_g)

    def gather_wait(j):
        pltpu.make_async_copy(node_hbm.at[pl.ds(0, CHUNK // 8)], stg, sem_g).wait()
        pltpu.make_async_copy(batch_hbm.at[pl.ds(0, CHUNK)],
                              idx3.at[lax.rem(j, 3)], sem_g).wait()

    def expand(buf):
        eb = exp.at[buf]

        def erow(i, carry):
            r = i * 2
            for rr in (r, r + 1):
                g = rr // 8
                r8 = lax.rem(rr, 8)
                vals = [stg[g, k // 8, r8, pl.ds((16 * k) % 128, 16)]
                        for k in range(NVEC)]
                for k in range(NVEC):
                    eb[rr, pl.ds(16 * k, 16)] = vals[k]
            return carry

        lax.fori_loop(0, CHUNK // 2, erow, 0)

    def scatter_start(j, buf, sem):
        ib = idx3.at[lax.rem(j, 3)]
        pltpu.async_copy(exp.at[buf], acc.at[ib], sem, add=True)
        pltpu.async_copy(ones_v, cnt.at[ib], sem, add=True)

    def scatter_drain(j, buf, sem):
        ib = idx3.at[lax.rem(j, 3)]
        pltpu.make_async_copy(exp.at[buf], acc.at[ib], sem).wait()
        pltpu.make_async_copy(ones_v, cnt.at[ib], sem).wait()

    @pl.when(n_mine > 0)
    def _():
        gather_start(0)

    def body(j, carry):
        buf = lax.rem(j, 2)

        @pl.when(j < n_mine)
        def _():
            @pl.when(j >= 2)
            def _():
                @pl.when(buf == 0)
                def _():
                    scatter_drain(j - 2, 0, sem_s0)

                @pl.when(buf == 1)
                def _():
                    scatter_drain(j - 2, 1, sem_s1)

            gather_wait(j)
            expand(buf)

            @pl.when(j + 1 < n_mine)
            def _():
                gather_start(j + 1)

            @pl.when(buf == 0)
            def _():
                scatter_start(j, 0, sem_s0)

            @pl.when(buf == 1)
            def _():
                scatter_start(j, 1, sem_s1)

        return carry

    lax.fori_loop(0, MAX_CHUNKS, body, 0)

    @pl.when(n_mine >= 2)
    def _():
        b2 = lax.rem(n_mine - 2, 2)

        @pl.when(b2 == 0)
        def _():
            scatter_drain(n_mine - 2, 0, sem_s0)

        @pl.when(b2 == 1)
        def _():
            scatter_drain(n_mine - 2, 1, sem_s1)

    @pl.when(n_mine >= 1)
    def _():
        b1 = lax.rem(n_mine - 1, 2)

        @pl.when(b1 == 0)
        def _():
            scatter_drain(n_mine - 1, 0, sem_s0)

        @pl.when(b1 == 1)
        def _():
            scatter_drain(n_mine - 1, 1, sem_s1)

    plsc.subcore_barrier()

    pltpu.sync_copy(acc.at[pl.ds(row0, G_PER_TILE)], exp.at[0, pl.ds(0, G_PER_TILE)])
    pltpu.sync_copy(exp.at[0, pl.ds(0, G_PER_TILE)], sums_out.at[c, pl.ds(row0, G_PER_TILE)])
    pltpu.sync_copy(cnt.at[pl.ds(row0, G_PER_TILE)], cntb_v)
    pltpu.sync_copy(cntb_v, cnts_out.at[c, pl.ds(row0, G_PER_TILE)])


def _tc_head_body(sums_ref, cnts_ref, w_ref, b_ref, out_ref):
    sums = sums_ref[0] + sums_ref[1]
    counts = cnts_ref[0, :, :1] + cnts_ref[1, :, :1]
    h = sums / jnp.clip(counts, 1.0, None)
    out_ref[...] = (
        lax.dot_general(
            h, w_ref[...],
            dimension_numbers=(((1,), (1,)), ((), ())),
            preferred_element_type=jnp.float32,
        )
        + b_ref[...]
    )


_tc_head = pl.pallas_call(
    _tc_head_body,
    out_shape=jax.ShapeDtypeStruct((NUM_G, NUM_T), jnp.float32),
)


@jax.jit
def kernel(node_rep, batch, W, b):
    batch32 = batch.astype(jnp.int32)
    node_t = _tc_tile(node_rep)
    w_p = jnp.pad(W, ((0, 0), (0, DIM_P - DIM)))
    zs = jnp.zeros((G_PER_TILE, DIM_P), jnp.float32)
    zc = jnp.zeros((G_PER_TILE, CNT_W), jnp.float32)
    ones = jnp.ones((CHUNK, CNT_W), jnp.float32)
    sums2, cnts2 = _sc_segment_sums(node_t, batch32, zs, zc, ones)
    return _tc_head(sums2, cnts2, w_p, b.reshape(1, NUM_T))


# speedup vs baseline: 2.3333x; 1.0055x over previous
import functools

import jax
import jax.numpy as jnp
from jax import lax
from jax.experimental import pallas as pl
from jax.experimental.pallas import tpu as pltpu
from jax.experimental.pallas import tpu_sc as plsc

N_NODES = 100000
NUM_G = 1024
DIM = 300
NUM_T = 128
CNT_W = 16
DIM_P = 384
NVEC = 19

CHUNK = 80
NUM_CHUNKS = N_NODES // CHUNK
NUM_WORKERS = 32
MAX_CHUNKS = -(-NUM_CHUNKS // NUM_WORKERS)
G_PER_TILE = NUM_G // 16

_mesh = plsc.VectorSubcoreMesh(core_axis_name="c", subcore_axis_name="s")


PAD_BLOCK = 5000


def _tc_tile_body(x_ref, o_ref):
    for t in range(2):
        o_ref[:, t] = x_ref[:, 128 * t:128 * (t + 1)].reshape(PAD_BLOCK // 8, 8, 128)
    tail = jnp.concatenate(
        [x_ref[:, 256:DIM], jnp.zeros((PAD_BLOCK, 384 - 256 - (DIM - 256)), jnp.float32)],
        axis=1,
    )
    o_ref[:, 2] = tail.reshape(PAD_BLOCK // 8, 8, 128)


_tc_tile = pl.pallas_call(
    _tc_tile_body,
    grid=(N_NODES // PAD_BLOCK,),
    in_specs=[pl.BlockSpec((PAD_BLOCK, DIM), lambda i: (i, 0))],
    out_specs=pl.BlockSpec((PAD_BLOCK // 8, 3, 8, 128), lambda i: (i, 0, 0, 0)),
    out_shape=jax.ShapeDtypeStruct((N_NODES // 8, 3, 8, 128), jnp.float32),
)


@functools.partial(
    pl.kernel,
    out_type=[
        jax.ShapeDtypeStruct((2, NUM_G, DIM_P), jnp.float32),
        jax.ShapeDtypeStruct((2, NUM_G, CNT_W), jnp.float32),
    ],
    mesh=_mesh,
    compiler_params=pltpu.CompilerParams(use_tc_tiling_on_sc=False),
    scratch_types=[
        pltpu.VMEM((CHUNK // 8, 3, 8, 128), jnp.float32),
        pltpu.VMEM((2, CHUNK, DIM_P), jnp.float32),
        pltpu.VMEM((3, CHUNK), jnp.int32),
        pltpu.VMEM((CHUNK, CNT_W), jnp.float32),
        pltpu.VMEM((G_PER_TILE, CNT_W), jnp.float32),
        pltpu.VMEM_SHARED((NUM_G, DIM_P), jnp.float32),
        pltpu.VMEM_SHARED((NUM_G, CNT_W), jnp.float32),
        pltpu.SemaphoreType.DMA,
        pltpu.SemaphoreType.DMA,
        pltpu.SemaphoreType.DMA,
    ],
)
def _sc_segment_sums(
    node_hbm, batch_hbm, zs_hbm, zc_hbm, ones_hbm,
    sums_out, cnts_out,
    stg, exp, idx3, ones_v, cntb_v, acc, cnt, sem_g, sem_s0, sem_s1,
):
    c = lax.axis_index("c")
    s = lax.axis_index("s")
    wid = s * 2 + c
    row0 = s * G_PER_TILE
    n_mine = (NUM_CHUNKS - wid + NUM_WORKERS - 1) // NUM_WORKERS

    pltpu.sync_copy(zs_hbm, exp.at[0, pl.ds(0, G_PER_TILE)])
    pltpu.sync_copy(exp.at[0, pl.ds(0, G_PER_TILE)], acc.at[pl.ds(row0, G_PER_TILE)])
    pltpu.sync_copy(zc_hbm, cntb_v)
    pltpu.sync_copy(cntb_v, cnt.at[pl.ds(row0, G_PER_TILE)])
    pltpu.sync_copy(ones_hbm, ones_v)
    plsc.subcore_barrier()

    def gather_start(j):
        cid = j * NUM_WORKERS + wid
        pltpu.async_copy(node_hbm.at[pl.ds(cid * (CHUNK // 8), CHUNK // 8)], stg, sem_g)
        pltpu.async_copy(batch_hbm.at[pl.ds(cid * CHUNK, CHUNK)],
                         idx3.at[lax.rem(j, 3)], sem_g)

    def gather_wait(j):
        pltpu.make_async_copy(node_hbm.at[pl.ds(0, CHUNK // 8)], stg, sem_g).wait()
        pltpu.make_async_copy(batch_hbm.at[pl.ds(0, CHUNK)],
                              idx3.at[lax.rem(j, 3)], sem_g).wait()

    def expand(buf):
        eb = exp.at[buf]

        def erow(i, carry):
            r = i * 2
            for rr in (r, r + 1):
                g = rr // 8
                r8 = lax.rem(rr, 8)
                vals = [stg[g, k // 8, r8, pl.ds((16 * k) % 128, 16)]
                        for k in range(NVEC)]
                for k in range(NVEC):
                    eb[rr, pl.ds(16 * k, 16)] = vals[k]
            return carry

        lax.fori_loop(0, CHUNK // 2, erow, 0)

    def scatter_start(j, buf, sem):
        ib = idx3.at[lax.rem(j, 3)]
        pltpu.async_copy(exp.at[buf], acc.at[ib], sem, add=True)
        pltpu.async_copy(ones_v, cnt.at[ib], sem, add=True)

    def scatter_drain(j, buf, sem):
        ib = idx3.at[lax.rem(j, 3)]
        pltpu.make_async_copy(exp.at[buf], acc.at[ib], sem).wait()
        pltpu.make_async_copy(ones_v, cnt.at[ib], sem).wait()

    @pl.when(n_mine > 0)
    def _():
        gather_start(0)

    def body(j, carry):
        buf = lax.rem(j, 2)

        @pl.when(j < n_mine)
        def _():
            @pl.when(j >= 2)
            def _():
                @pl.when(buf == 0)
                def _():
                    scatter_drain(j - 2, 0, sem_s0)

                @pl.when(buf == 1)
                def _():
                    scatter_drain(j - 2, 1, sem_s1)

            gather_wait(j)
            expand(buf)

            @pl.when(j + 1 < n_mine)
            def _():
                gather_start(j + 1)

            @pl.when(buf == 0)
            def _():
                scatter_start(j, 0, sem_s0)

            @pl.when(buf == 1)
            def _():
                scatter_start(j, 1, sem_s1)

        return carry

    lax.fori_loop(0, MAX_CHUNKS, body, 0)

    @pl.when(n_mine >= 2)
    def _():
        b2 = lax.rem(n_mine - 2, 2)

        @pl.when(b2 == 0)
        def _():
            scatter_drain(n_mine - 2, 0, sem_s0)

        @pl.when(b2 == 1)
        def _():
            scatter_drain(n_mine - 2, 1, sem_s1)

    @pl.when(n_mine >= 1)
    def _():
        b1 = lax.rem(n_mine - 1, 2)

        @pl.when(b1 == 0)
        def _():
            scatter_drain(n_mine - 1, 0, sem_s0)

        @pl.when(b1 == 1)
        def _():
            scatter_drain(n_mine - 1, 1, sem_s1)

    plsc.subcore_barrier()

    pltpu.sync_copy(acc.at[pl.ds(row0, G_PER_TILE)], exp.at[0, pl.ds(0, G_PER_TILE)])
    pltpu.sync_copy(exp.at[0, pl.ds(0, G_PER_TILE)], sums_out.at[c, pl.ds(row0, G_PER_TILE)])
    pltpu.sync_copy(cnt.at[pl.ds(row0, G_PER_TILE)], cntb_v)
    pltpu.sync_copy(cntb_v, cnts_out.at[c, pl.ds(row0, G_PER_TILE)])


def _tc_head_body(sums_ref, cnts_ref, w_ref, b_ref, out_ref):
    sums = sums_ref[0] + sums_ref[1]
    counts = cnts_ref[0, :, :1] + cnts_ref[1, :, :1]
    h = sums / jnp.clip(counts, 1.0, None)
    out_ref[...] = (
        lax.dot_general(
            h, w_ref[...],
            dimension_numbers=(((1,), (1,)), ((), ())),
            preferred_element_type=jnp.float32,
        )
        + b_ref[...]
    )


_tc_head = pl.pallas_call(
    _tc_head_body,
    out_shape=jax.ShapeDtypeStruct((NUM_G, NUM_T), jnp.float32),
)


@jax.jit
def kernel(node_rep, batch, W, b):
    batch32 = batch.astype(jnp.int32)
    node_t = _tc_tile(node_rep)
    w_p = jnp.pad(W, ((0, 0), (0, DIM_P - DIM)))
    zs = jnp.zeros((G_PER_TILE, DIM_P), jnp.float32)
    zc = jnp.zeros((G_PER_TILE, CNT_W), jnp.float32)
    ones = jnp.ones((CHUNK, CNT_W), jnp.float32)
    sums2, cnts2 = _sc_segment_sums(node_t, batch32, zs, zc, ones)
    return _tc_head(sums2, cnts2, w_p, b.reshape(1, NUM_T))


# speedup vs baseline: 2.3392x; 1.0025x over previous
import functools

import jax
import jax.numpy as jnp
from jax import lax
from jax.experimental import pallas as pl
from jax.experimental.pallas import tpu as pltpu
from jax.experimental.pallas import tpu_sc as plsc

N_NODES = 100000
NUM_G = 1024
DIM = 300
NUM_T = 128
CNT_W = 16
DIM_P = 384
NVEC = 19

CHUNK = 80
NUM_CHUNKS = N_NODES // CHUNK
NUM_WORKERS = 32
MAX_CHUNKS = -(-NUM_CHUNKS // NUM_WORKERS)
G_PER_TILE = NUM_G // 16

_mesh = plsc.VectorSubcoreMesh(core_axis_name="c", subcore_axis_name="s")


PAD_BLOCK = 10000


def _tc_tile_body(x_ref, o_ref):
    for t in range(2):
        o_ref[:, t] = x_ref[:, 128 * t:128 * (t + 1)].reshape(PAD_BLOCK // 8, 8, 128)
    tail = jnp.concatenate(
        [x_ref[:, 256:DIM], jnp.zeros((PAD_BLOCK, 384 - 256 - (DIM - 256)), jnp.float32)],
        axis=1,
    )
    o_ref[:, 2] = tail.reshape(PAD_BLOCK // 8, 8, 128)


_tc_tile = pl.pallas_call(
    _tc_tile_body,
    grid=(N_NODES // PAD_BLOCK,),
    in_specs=[pl.BlockSpec((PAD_BLOCK, DIM), lambda i: (i, 0))],
    out_specs=pl.BlockSpec((PAD_BLOCK // 8, 3, 8, 128), lambda i: (i, 0, 0, 0)),
    out_shape=jax.ShapeDtypeStruct((N_NODES // 8, 3, 8, 128), jnp.float32),
)


@functools.partial(
    pl.kernel,
    out_type=[
        jax.ShapeDtypeStruct((2, NUM_G, DIM_P), jnp.float32),
        jax.ShapeDtypeStruct((2, NUM_G, CNT_W), jnp.float32),
    ],
    mesh=_mesh,
    compiler_params=pltpu.CompilerParams(use_tc_tiling_on_sc=False),
    scratch_types=[
        pltpu.VMEM((CHUNK // 8, 3, 8, 128), jnp.float32),
        pltpu.VMEM((2, CHUNK, DIM_P), jnp.float32),
        pltpu.VMEM((3, CHUNK), jnp.int32),
        pltpu.VMEM((CHUNK, CNT_W), jnp.float32),
        pltpu.VMEM((G_PER_TILE, CNT_W), jnp.float32),
        pltpu.VMEM_SHARED((NUM_G, DIM_P), jnp.float32),
        pltpu.VMEM_SHARED((NUM_G, CNT_W), jnp.float32),
        pltpu.SemaphoreType.DMA,
        pltpu.SemaphoreType.DMA,
        pltpu.SemaphoreType.DMA,
    ],
)
def _sc_segment_sums(
    node_hbm, batch_hbm, zs_hbm, zc_hbm, ones_hbm,
    sums_out, cnts_out,
    stg, exp, idx3, ones_v, cntb_v, acc, cnt, sem_g, sem_s0, sem_s1,
):
    c = lax.axis_index("c")
    s = lax.axis_index("s")
    wid = s * 2 + c
    row0 = s * G_PER_TILE
    n_mine = (NUM_CHUNKS - wid + NUM_WORKERS - 1) // NUM_WORKERS

    pltpu.sync_copy(zs_hbm, exp.at[0, pl.ds(0, G_PER_TILE)])
    pltpu.sync_copy(exp.at[0, pl.ds(0, G_PER_TILE)], acc.at[pl.ds(row0, G_PER_TILE)])
    pltpu.sync_copy(zc_hbm, cntb_v)
    pltpu.sync_copy(cntb_v, cnt.at[pl.ds(row0, G_PER_TILE)])
    pltpu.sync_copy(ones_hbm, ones_v)
    plsc.subcore_barrier()

    def gather_start(j):
        cid = j * NUM_WORKERS + wid
        pltpu.async_copy(node_hbm.at[pl.ds(cid * (CHUNK // 8), CHUNK // 8)], stg, sem_g)
        pltpu.async_copy(batch_hbm.at[pl.ds(cid * CHUNK, CHUNK)],
                         idx3.at[lax.rem(j, 3)], sem_g)

    def gather_wait(j):
        pltpu.make_async_copy(node_hbm.at[pl.ds(0, CHUNK // 8)], stg, sem_g).wait()
        pltpu.make_async_copy(batch_hbm.at[pl.ds(0, CHUNK)],
                              idx3.at[lax.rem(j, 3)], sem_g).wait()

    def expand(buf):
        eb = exp.at[buf]

        def erow(i, carry):
            r = i * 2
            for rr in (r, r + 1):
                g = rr // 8
                r8 = lax.rem(rr, 8)
                vals = [stg[g, k // 8, r8, pl.ds((16 * k) % 128, 16)]
                        for k in range(NVEC)]
                for k in range(NVEC):
                    eb[rr, pl.ds(16 * k, 16)] = vals[k]
            return carry

        lax.fori_loop(0, CHUNK // 2, erow, 0)

    def scatter_start(j, buf, sem):
        ib = idx3.at[lax.rem(j, 3)]
        pltpu.async_copy(exp.at[buf], acc.at[ib], sem, add=True)
        pltpu.async_copy(ones_v, cnt.at[ib], sem, add=True)

    def scatter_drain(j, buf, sem):
        ib = idx3.at[lax.rem(j, 3)]
        pltpu.make_async_copy(exp.at[buf], acc.at[ib], sem).wait()
        pltpu.make_async_copy(ones_v, cnt.at[ib], sem).wait()

    @pl.when(n_mine > 0)
    def _():
        gather_start(0)

    def body(j, carry):
        buf = lax.rem(j, 2)

        @pl.when(j < n_mine)
        def _():
            @pl.when(j >= 2)
            def _():
                @pl.when(buf == 0)
                def _():
                    scatter_drain(j - 2, 0, sem_s0)

                @pl.when(buf == 1)
                def _():
                    scatter_drain(j - 2, 1, sem_s1)

            gather_wait(j)
            expand(buf)

            @pl.when(j + 1 < n_mine)
            def _():
                gather_start(j + 1)

            @pl.when(buf == 0)
            def _():
                scatter_start(j, 0, sem_s0)

            @pl.when(buf == 1)
            def _():
                scatter_start(j, 1, sem_s1)

        return carry

    lax.fori_loop(0, MAX_CHUNKS, body, 0)

    @pl.when(n_mine >= 2)
    def _():
        b2 = lax.rem(n_mine - 2, 2)

        @pl.when(b2 == 0)
        def _():
            scatter_drain(n_mine - 2, 0, sem_s0)

        @pl.when(b2 == 1)
        def _():
            scatter_drain(n_mine - 2, 1, sem_s1)

    @pl.when(n_mine >= 1)
    def _():
        b1 = lax.rem(n_mine - 1, 2)

        @pl.when(b1 == 0)
        def _():
            scatter_drain(n_mine - 1, 0, sem_s0)

        @pl.when(b1 == 1)
        def _():
            scatter_drain(n_mine - 1, 1, sem_s1)

    plsc.subcore_barrier()

    pltpu.sync_copy(acc.at[pl.ds(row0, G_PER_TILE)], exp.at[0, pl.ds(0, G_PER_TILE)])
    pltpu.sync_copy(exp.at[0, pl.ds(0, G_PER_TILE)], sums_out.at[c, pl.ds(row0, G_PER_TILE)])
    pltpu.sync_copy(cnt.at[pl.ds(row0, G_PER_TILE)], cntb_v)
    pltpu.sync_copy(cntb_v, cnts_out.at[c, pl.ds(row0, G_PER_TILE)])


def _tc_head_body(sums_ref, cnts_ref, w_ref, b_ref, out_ref):
    sums = sums_ref[0] + sums_ref[1]
    counts = cnts_ref[0, :, :1] + cnts_ref[1, :, :1]
    h = sums / jnp.clip(counts, 1.0, None)
    out_ref[...] = (
        lax.dot_general(
            h, w_ref[...],
            dimension_numbers=(((1,), (1,)), ((), ())),
            preferred_element_type=jnp.float32,
        )
        + b_ref[...]
    )


_tc_head = pl.pallas_call(
    _tc_head_body,
    out_shape=jax.ShapeDtypeStruct((NUM_G, NUM_T), jnp.float32),
)


@jax.jit
def kernel(node_rep, batch, W, b):
    batch32 = batch.astype(jnp.int32)
    node_t = _tc_tile(node_rep)
    w_p = jnp.pad(W, ((0, 0), (0, DIM_P - DIM)))
    zs = jnp.zeros((G_PER_TILE, DIM_P), jnp.float32)
    zc = jnp.zeros((G_PER_TILE, CNT_W), jnp.float32)
    ones = jnp.ones((CHUNK, CNT_W), jnp.float32)
    sums2, cnts2 = _sc_segment_sums(node_t, batch32, zs, zc, ones)
    return _tc_head(sums2, cnts2, w_p, b.reshape(1, NUM_T))
